# Initial kernel scaffold; baseline (speedup 1.0000x reference)
import jax
import jax.numpy as jnp
from jax.experimental import pallas as pl


def kernel(x, n_id, res_n_id, edge_src, edge_dst, W, b):
    raise NotImplementedError("write your pallas kernel here")


# speedup vs baseline: 28.5181x; 28.5181x over previous
import functools

import jax
import jax.numpy as jnp
from jax import lax
from jax.experimental import pallas as pl
from jax.experimental.pallas import tpu as pltpu
from jax.experimental.pallas import tpu_sc as plsc

N_SRC = 10000
N_DST = 2048
E = 320000
D = 128

NC = 2
NS = 16
NW = NC * NS

B_PAD = 10240
GPW = B_PAD // NW
GK = 64

EPW = E // NW
EK = 125
NCH = EPW // EK

HTOT = 12288
SW = HTOT // NS

_mesh = plsc.VectorSubcoreMesh(core_axis_name="c", subcore_axis_name="s")


def _wid():
    return lax.axis_index("s") * NC + lax.axis_index("c")


@functools.partial(
    pl.kernel,
    out_type=jax.ShapeDtypeStruct((B_PAD, D), jnp.float32),
    mesh=_mesh,
    scratch_types=[
        pltpu.VMEM((GK,), jnp.int32),
        pltpu.VMEM((GK, D), jnp.float32),
        pltpu.SemaphoreType.DMA,
    ],
)
def _gather_rows(x_hbm, nid_hbm, out_hbm, idx_v, rows_v, sem):
    base = _wid() * GPW
    for j in range(GPW // GK):
        off = base + j * GK
        pltpu.sync_copy(nid_hbm.at[pl.ds(off, GK)], idx_v)
        pltpu.async_copy(x_hbm.at[idx_v], rows_v, sem).wait()
        pltpu.sync_copy(rows_v, out_hbm.at[pl.ds(off, GK)])


@functools.partial(
    pl.kernel,
    out_type=[jax.ShapeDtypeStruct((HTOT,), jnp.float32),
              jax.ShapeDtypeStruct((HTOT,), jnp.float32)],
    mesh=_mesh,
    scratch_types=[
        pltpu.VMEM((EPW,), jnp.int32),
        pltpu.VMEM((EPW,), jnp.int32),
        pltpu.VMEM((HTOT,), jnp.float32),
        pltpu.VMEM((SW,), jnp.float32),
        pltpu.VMEM((SW,), jnp.float32),
        pltpu.VMEM_SHARED((NS * HTOT,), jnp.float32),
    ],
    compiler_params=pltpu.CompilerParams(needs_layout_passes=False),
)
def _histograms(esrc_hbm, edst_hbm, out0_hbm, out1_hbm, sidx_v, didx_v,
                hist_v, acc_v, tbuf_v, hist_sh):
    cid = lax.axis_index("c")
    sid = lax.axis_index("s")
    wid = _wid()
    zeros16 = jnp.zeros((16,), jnp.float32)
    ones16 = jnp.ones((16,), jnp.float32)

    def zero_body(j, carry):
        hist_v[pl.ds(pl.multiple_of(j * 16, 16), 16)] = zeros16
        return carry

    lax.fori_loop(0, HTOT // 16, zero_body, None)

    pltpu.sync_copy(esrc_hbm.at[pl.ds(wid * EPW, EPW)], sidx_v)
    pltpu.sync_copy(edst_hbm.at[pl.ds(wid * EPW, EPW)], didx_v)

    def scat_body(i, carry):
        sl = pl.ds(pl.multiple_of(i * 16, 16), 16)
        plsc.addupdate_scatter(hist_v, [sidx_v[sl]], ones16)
        plsc.addupdate_scatter(hist_v, [didx_v[sl] + N_SRC], ones16)
        return carry

    lax.fori_loop(0, EPW // 16, scat_body, None)

    pltpu.sync_copy(hist_v, hist_sh.at[pl.ds(sid * HTOT, HTOT)])
    plsc.subcore_barrier()

    def zacc_body(j, carry):
        acc_v[pl.ds(pl.multiple_of(j * 16, 16), 16)] = zeros16
        return carry

    lax.fori_loop(0, SW // 16, zacc_body, None)

    def red_body(t, carry):
        pltpu.sync_copy(
            hist_sh.at[pl.ds(pl.multiple_of(t * HTOT + sid * SW, 128), SW)],
            tbuf_v)

        def add_body(j, c2):
            sl = pl.ds(pl.multiple_of(j * 16, 16), 16)
            acc_v[sl] = acc_v[sl] + tbuf_v[sl]
            return c2

        lax.fori_loop(0, SW // 16, add_body, None)
        return carry

    lax.fori_loop(0, NS, red_body, None)

    @pl.when(cid == 0)
    def _():
        pltpu.sync_copy(acc_v, out0_hbm.at[pl.ds(sid * SW, SW)])

    @pl.when(cid == 1)
    def _():
        pltpu.sync_copy(acc_v, out1_hbm.at[pl.ds(sid * SW, SW)])


def _matmul_body(x_ref, w_ref, d0_ref, d1_ref, o_ref):
    deg = d0_ref[...] + d1_ref[...]
    f = lax.rsqrt(jnp.maximum(deg, 1.0))
    h = jnp.dot(x_ref[...], w_ref[...], preferred_element_type=jnp.float32,
                precision=lax.Precision.HIGHEST)
    o_ref[...] = h * f


def _matmul(x_g, W, ds0, ds1):
    blk = 512
    grid = B_PAD // blk
    return pl.pallas_call(
        _matmul_body,
        grid=(grid,),
        in_specs=[
            pl.BlockSpec((blk, D), lambda i: (i, 0)),
            pl.BlockSpec((D, D), lambda i: (0, 0)),
            pl.BlockSpec((blk, 1), lambda i: (i, 0)),
            pl.BlockSpec((blk, 1), lambda i: (i, 0)),
        ],
        out_specs=pl.BlockSpec((blk, D), lambda i: (i, 0)),
        out_shape=jax.ShapeDtypeStruct((B_PAD, D), jnp.float32),
    )(x_g, W, ds0, ds1)


@functools.partial(
    pl.kernel,
    out_type=jax.ShapeDtypeStruct((NC, N_DST, D), jnp.float32),
    mesh=_mesh,
    scratch_types=[
        pltpu.VMEM((NCH, EK), jnp.int32),
        pltpu.VMEM((NCH, EK), jnp.int32),
        pltpu.VMEM((EK, D), jnp.float32),
        pltpu.VMEM((EK, D), jnp.float32),
        pltpu.SemaphoreType.DMA,
        pltpu.SemaphoreType.DMA,
        pltpu.VMEM_SHARED((N_DST, D), jnp.float32),
    ],
)
def _aggregate(h_hbm, esrc_hbm, edst_hbm, zero_hbm, out_hbm,
               sidx_v, didx_v, rows0_v, rows1_v, sem0, sem1, agg_sh):
    cid = lax.axis_index("c")
    sid = lax.axis_index("s")
    wid = _wid()
    rpt = N_DST // NS

    pltpu.sync_copy(zero_hbm.at[pl.ds(sid * rpt, rpt), :],
                    agg_sh.at[pl.ds(sid * rpt, rpt), :])
    plsc.subcore_barrier()

    pltpu.sync_copy(esrc_hbm.at[pl.ds(wid * NCH, NCH), :], sidx_v)
    pltpu.sync_copy(edst_hbm.at[pl.ds(wid * NCH, NCH), :], didx_v)

    def pair_body(g, carry):
        ja = 2 * g
        jb = ja + 1
        d0 = pltpu.async_copy(h_hbm.at[sidx_v.at[ja]], rows0_v, sem0)
        d1 = pltpu.async_copy(h_hbm.at[sidx_v.at[jb]], rows1_v, sem1)
        d0.wait()
        pltpu.sync_copy(rows0_v, agg_sh.at[didx_v.at[ja]], add=True)
        d1.wait()
        pltpu.sync_copy(rows1_v, agg_sh.at[didx_v.at[jb]], add=True)
        return carry

    lax.fori_loop(0, NCH // 2, pair_body, None)

    plsc.subcore_barrier()
    pltpu.sync_copy(agg_sh.at[pl.ds(sid * rpt, rpt), :],
                    out_hbm.at[cid, pl.ds(sid * rpt, rpt), :])


def _final_body(p0_ref, p1_ref, d0_ref, d1_ref, b_ref, o_ref):
    g = lax.rsqrt(jnp.maximum(d0_ref[...] + d1_ref[...], 1.0))
    a = (p0_ref[...] + p1_ref[...]) * g + b_ref[...]
    o_ref[...] = jnp.where(a > 0, a, jnp.exp(jnp.minimum(a, 0.0)) - 1.0)


def _finalize(p0, p1, dd0, dd1, b2):
    return pl.pallas_call(
        _final_body,
        grid=(1,),
        in_specs=[
            pl.BlockSpec((N_DST, D), lambda i: (0, 0)),
            pl.BlockSpec((N_DST, D), lambda i: (0, 0)),
            pl.BlockSpec((N_DST, 1), lambda i: (0, 0)),
            pl.BlockSpec((N_DST, 1), lambda i: (0, 0)),
            pl.BlockSpec((1, D), lambda i: (0, 0)),
        ],
        out_specs=pl.BlockSpec((N_DST, D), lambda i: (0, 0)),
        out_shape=jax.ShapeDtypeStruct((N_DST, D), jnp.float32),
    )(p0, p1, dd0, dd1, b2)


def kernel(x, n_id, res_n_id, edge_src, edge_dst, W, b):
    del res_n_id
    nid_pad = jnp.concatenate(
        [n_id, jnp.zeros((B_PAD - N_SRC,), jnp.int32)])
    esrc_r = edge_src.reshape(NW * NCH, EK)
    edst_r = edge_dst.reshape(NW * NCH, EK)

    x_g = _gather_rows(x, nid_pad)
    hist0, hist1 = _histograms(edge_src, edge_dst)

    ds0 = hist0[:B_PAD].reshape(B_PAD, 1)
    ds1 = hist1[:B_PAD].reshape(B_PAD, 1)
    h = _matmul(x_g, W, ds0, ds1)

    zeros2d = jnp.zeros((N_DST, D), jnp.float32)
    parts = _aggregate(h, esrc_r, edst_r, zeros2d)

    dd0 = hist0[N_SRC:N_SRC + N_DST].reshape(N_DST, 1)
    dd1 = hist1[N_SRC:N_SRC + N_DST].reshape(N_DST, 1)
    return _finalize(parts[0], parts[1], dd0, dd1, b.reshape(1, D))


# speedup vs baseline: 30.6997x; 1.0765x over previous
import functools

import jax
import jax.numpy as jnp
from jax import lax
from jax.experimental import pallas as pl
from jax.experimental.pallas import tpu as pltpu
from jax.experimental.pallas import tpu_sc as plsc

N_SRC = 10000
N_DST = 2048
E = 320000
D = 128

NC = 2
NS = 16
NW = NC * NS

B_PAD = 10240
GPW = B_PAD // NW
GK = 64

EPW = E // NW
EK = 125
NCH = EPW // EK

HTOT = 12288
SW = HTOT // NS

_mesh = plsc.VectorSubcoreMesh(core_axis_name="c", subcore_axis_name="s")


def _wid():
    return lax.axis_index("s") * NC + lax.axis_index("c")


@functools.partial(
    pl.kernel,
    out_type=[jax.ShapeDtypeStruct((B_PAD, D), jnp.float32),
              jax.ShapeDtypeStruct((HTOT,), jnp.float32),
              jax.ShapeDtypeStruct((HTOT,), jnp.float32)],
    mesh=_mesh,
    scratch_types=[
        pltpu.VMEM((GPW,), jnp.int32),
        [pltpu.VMEM((GK, D), jnp.float32) for _ in range(GPW // GK)],
        [pltpu.SemaphoreType.DMA for _ in range(GPW // GK)],
        pltpu.VMEM((EPW,), jnp.int32),
        pltpu.VMEM((EPW,), jnp.int32),
        pltpu.SemaphoreType.DMA,
        pltpu.SemaphoreType.DMA,
        pltpu.VMEM((HTOT,), jnp.float32),
        pltpu.VMEM((SW,), jnp.float32),
        pltpu.VMEM((SW,), jnp.float32),
        pltpu.VMEM_SHARED((NS * HTOT,), jnp.float32),
    ],
    compiler_params=pltpu.CompilerParams(needs_layout_passes=False),
)
def _gather_hist(x_hbm, nid_hbm, esrc_hbm, edst_hbm,
                 out_hbm, out0_hbm, out1_hbm,
                 gidx_v, rows_bufs, gsems, sidx_v, didx_v, esem0, esem1,
                 hist_v, acc_v, tbuf_v, hist_sh):
    cid = lax.axis_index("c")
    sid = lax.axis_index("s")
    wid = _wid()
    zeros16 = jnp.zeros((16,), jnp.float32)
    ones16 = jnp.ones((16,), jnp.float32)
    nchunk = GPW // GK
    base = wid * GPW

    pltpu.sync_copy(nid_hbm.at[pl.ds(base, GPW)], gidx_v)
    gds = [pltpu.async_copy(x_hbm.at[gidx_v.at[pl.ds(j * GK, GK)]],
                            rows_bufs[j], gsems[j])
           for j in range(nchunk)]
    ed0 = pltpu.async_copy(esrc_hbm.at[pl.ds(wid * EPW, EPW)], sidx_v, esem0)
    ed1 = pltpu.async_copy(edst_hbm.at[pl.ds(wid * EPW, EPW)], didx_v, esem1)

    def zero_body(j, carry):
        hist_v[pl.ds(pl.multiple_of(j * 16, 16), 16)] = zeros16
        return carry

    lax.fori_loop(0, HTOT // 16, zero_body, None)

    for j in range(nchunk):
        gds[j].wait()
        pltpu.sync_copy(rows_bufs[j], out_hbm.at[pl.ds(base + j * GK, GK)])
    ed0.wait()
    ed1.wait()

    def scat_body(i, carry):
        sl = pl.ds(pl.multiple_of(i * 16, 16), 16)
        plsc.addupdate_scatter(hist_v, [sidx_v[sl]], ones16)
        plsc.addupdate_scatter(hist_v, [didx_v[sl] + N_SRC], ones16)
        return carry

    lax.fori_loop(0, EPW // 16, scat_body, None)

    pltpu.sync_copy(hist_v, hist_sh.at[pl.ds(sid * HTOT, HTOT)])
    plsc.subcore_barrier()

    def zacc_body(j, carry):
        acc_v[pl.ds(pl.multiple_of(j * 16, 16), 16)] = zeros16
        return carry

    lax.fori_loop(0, SW // 16, zacc_body, None)

    def red_body(t, carry):
        pltpu.sync_copy(
            hist_sh.at[pl.ds(pl.multiple_of(t * HTOT + sid * SW, 128), SW)],
            tbuf_v)

        def add_body(j, c2):
            sl = pl.ds(pl.multiple_of(j * 16, 16), 16)
            acc_v[sl] = acc_v[sl] + tbuf_v[sl]
            return c2

        lax.fori_loop(0, SW // 16, add_body, None)
        return carry

    lax.fori_loop(0, NS, red_body, None)

    @pl.when(cid == 0)
    def _():
        pltpu.sync_copy(acc_v, out0_hbm.at[pl.ds(sid * SW, SW)])

    @pl.when(cid == 1)
    def _():
        pltpu.sync_copy(acc_v, out1_hbm.at[pl.ds(sid * SW, SW)])


def _matmul_body(x_ref, w_ref, d0_ref, d1_ref, o_ref):
    deg = d0_ref[...] + d1_ref[...]
    f = lax.rsqrt(jnp.maximum(deg, 1.0))
    h = jnp.dot(x_ref[...], w_ref[...], preferred_element_type=jnp.float32,
                precision=lax.Precision.HIGHEST)
    o_ref[...] = h * f


def _matmul(x_g, W, ds0, ds1):
    blk = 512
    grid = B_PAD // blk
    return pl.pallas_call(
        _matmul_body,
        grid=(grid,),
        in_specs=[
            pl.BlockSpec((blk, D), lambda i: (i, 0)),
            pl.BlockSpec((D, D), lambda i: (0, 0)),
            pl.BlockSpec((blk, 1), lambda i: (i, 0)),
            pl.BlockSpec((blk, 1), lambda i: (i, 0)),
        ],
        out_specs=pl.BlockSpec((blk, D), lambda i: (i, 0)),
        out_shape=jax.ShapeDtypeStruct((B_PAD, D), jnp.float32),
    )(x_g, W, ds0, ds1)


@functools.partial(
    pl.kernel,
    out_type=jax.ShapeDtypeStruct((NC, N_DST, D), jnp.float32),
    mesh=_mesh,
    scratch_types=[
        pltpu.VMEM((NCH, EK), jnp.int32),
        pltpu.VMEM((NCH, EK), jnp.int32),
        [pltpu.VMEM((EK, D), jnp.float32) for _ in range(4)],
        [pltpu.SemaphoreType.DMA for _ in range(4)],
        [pltpu.SemaphoreType.DMA for _ in range(4)],
        pltpu.VMEM_SHARED((N_DST, D), jnp.float32),
    ],
)
def _aggregate(h_hbm, esrc_hbm, edst_hbm, zero_hbm, out_hbm,
               sidx_v, didx_v, rows_bufs, gsems, ssems, agg_sh):
    cid = lax.axis_index("c")
    sid = lax.axis_index("s")
    wid = _wid()
    rpt = N_DST // NS

    pltpu.sync_copy(zero_hbm.at[pl.ds(sid * rpt, rpt), :],
                    agg_sh.at[pl.ds(sid * rpt, rpt), :])
    plsc.subcore_barrier()

    pltpu.sync_copy(esrc_hbm.at[pl.ds(wid * NCH, NCH), :], sidx_v)
    pltpu.sync_copy(edst_hbm.at[pl.ds(wid * NCH, NCH), :], didx_v)

    def quad_body(g, carry):
        j0 = 4 * g
        gds = [pltpu.async_copy(h_hbm.at[sidx_v.at[j0 + b]],
                                rows_bufs[b], gsems[b])
               for b in range(4)]
        sds = []
        for b in range(4):
            gds[b].wait()
            sds.append(pltpu.async_copy(rows_bufs[b],
                                        agg_sh.at[didx_v.at[j0 + b]],
                                        ssems[b], add=True))
        for b in range(4):
            sds[b].wait()
        return carry

    lax.fori_loop(0, NCH // 4, quad_body, None)

    plsc.subcore_barrier()
    pltpu.sync_copy(agg_sh.at[pl.ds(sid * rpt, rpt), :],
                    out_hbm.at[cid, pl.ds(sid * rpt, rpt), :])


def _final_body(p0_ref, p1_ref, d0_ref, d1_ref, b_ref, o_ref):
    g = lax.rsqrt(jnp.maximum(d0_ref[...] + d1_ref[...], 1.0))
    a = (p0_ref[...] + p1_ref[...]) * g + b_ref[...]
    o_ref[...] = jnp.where(a > 0, a, jnp.exp(jnp.minimum(a, 0.0)) - 1.0)


def _finalize(p0, p1, dd0, dd1, b2):
    return pl.pallas_call(
        _final_body,
        grid=(1,),
        in_specs=[
            pl.BlockSpec((N_DST, D), lambda i: (0, 0)),
            pl.BlockSpec((N_DST, D), lambda i: (0, 0)),
            pl.BlockSpec((N_DST, 1), lambda i: (0, 0)),
            pl.BlockSpec((N_DST, 1), lambda i: (0, 0)),
            pl.BlockSpec((1, D), lambda i: (0, 0)),
        ],
        out_specs=pl.BlockSpec((N_DST, D), lambda i: (0, 0)),
        out_shape=jax.ShapeDtypeStruct((N_DST, D), jnp.float32),
    )(p0, p1, dd0, dd1, b2)


def kernel(x, n_id, res_n_id, edge_src, edge_dst, W, b):
    del res_n_id
    nid_pad = jnp.concatenate(
        [n_id, jnp.zeros((B_PAD - N_SRC,), jnp.int32)])
    esrc_r = edge_src.reshape(NW * NCH, EK)
    edst_r = edge_dst.reshape(NW * NCH, EK)

    x_g, hist0, hist1 = _gather_hist(x, nid_pad, edge_src, edge_dst)

    ds0 = hist0[:B_PAD].reshape(B_PAD, 1)
    ds1 = hist1[:B_PAD].reshape(B_PAD, 1)
    h = _matmul(x_g, W, ds0, ds1)

    zeros2d = jnp.zeros((N_DST, D), jnp.float32)
    parts = _aggregate(h, esrc_r, edst_r, zeros2d)

    dd0 = hist0[N_SRC:N_SRC + N_DST].reshape(N_DST, 1)
    dd1 = hist1[N_SRC:N_SRC + N_DST].reshape(N_DST, 1)
    return _finalize(parts[0], parts[1], dd0, dd1, b.reshape(1, D))


# speedup vs baseline: 35.4777x; 1.1556x over previous
import functools

import jax
import jax.numpy as jnp
from jax import lax
from jax.experimental import pallas as pl
from jax.experimental.pallas import tpu as pltpu
from jax.experimental.pallas import tpu_sc as plsc

N_SRC = 10000
N_DST = 2048
E = 320000
D = 128

NC = 2
NS = 16
NW = NC * NS

B_PAD = 10240
GPW = B_PAD // NW
GK = 64

EPW = E // NW
EK = 80
NCH = EPW // EK

HTOT = 12288
SW = HTOT // NS

_mesh = plsc.VectorSubcoreMesh(core_axis_name="c", subcore_axis_name="s")


def _wid():
    return lax.axis_index("s") * NC + lax.axis_index("c")


@functools.partial(
    pl.kernel,
    out_type=[jax.ShapeDtypeStruct((B_PAD, D), jnp.float32),
              jax.ShapeDtypeStruct((HTOT,), jnp.float32),
              jax.ShapeDtypeStruct((HTOT,), jnp.float32)],
    mesh=_mesh,
    scratch_types=[
        pltpu.VMEM((GPW,), jnp.int32),
        [pltpu.VMEM((GK, D), jnp.float32) for _ in range(GPW // GK)],
        [pltpu.SemaphoreType.DMA for _ in range(GPW // GK)],
        pltpu.VMEM((EPW,), jnp.int32),
        pltpu.VMEM((EPW,), jnp.int32),
        pltpu.SemaphoreType.DMA,
        pltpu.SemaphoreType.DMA,
        pltpu.VMEM((HTOT,), jnp.float32),
        pltpu.VMEM((SW,), jnp.float32),
        pltpu.VMEM((SW,), jnp.float32),
        pltpu.VMEM_SHARED((NS * HTOT,), jnp.float32),
    ],
    compiler_params=pltpu.CompilerParams(needs_layout_passes=False),
)
def _gather_hist(x_hbm, nid_hbm, esrc_hbm, edst_hbm,
                 out_hbm, out0_hbm, out1_hbm,
                 gidx_v, rows_bufs, gsems, sidx_v, didx_v, esem0, esem1,
                 hist_v, acc_v, tbuf_v, hist_sh):
    cid = lax.axis_index("c")
    sid = lax.axis_index("s")
    wid = _wid()
    zeros16 = jnp.zeros((16,), jnp.float32)
    ones16 = jnp.ones((16,), jnp.float32)
    nchunk = GPW // GK
    base = wid * GPW

    pltpu.sync_copy(nid_hbm.at[pl.ds(base, GPW)], gidx_v)
    gds = [pltpu.async_copy(x_hbm.at[gidx_v.at[pl.ds(j * GK, GK)]],
                            rows_bufs[j], gsems[j])
           for j in range(nchunk)]
    ed0 = pltpu.async_copy(esrc_hbm.at[pl.ds(wid * EPW, EPW)], sidx_v, esem0)
    ed1 = pltpu.async_copy(edst_hbm.at[pl.ds(wid * EPW, EPW)], didx_v, esem1)

    def zero_body(j, carry):
        hist_v[pl.ds(pl.multiple_of(j * 16, 16), 16)] = zeros16
        return carry

    lax.fori_loop(0, HTOT // 16, zero_body, None)

    for j in range(nchunk):
        gds[j].wait()
        pltpu.sync_copy(rows_bufs[j], out_hbm.at[pl.ds(base + j * GK, GK)])
    ed0.wait()
    ed1.wait()

    def scat_body(i, carry):
        sl = pl.ds(pl.multiple_of(i * 16, 16), 16)
        plsc.addupdate_scatter(hist_v, [sidx_v[sl]], ones16)
        plsc.addupdate_scatter(hist_v, [didx_v[sl] + N_SRC], ones16)
        return carry

    lax.fori_loop(0, EPW // 16, scat_body, None)

    pltpu.sync_copy(hist_v, hist_sh.at[pl.ds(sid * HTOT, HTOT)])
    plsc.subcore_barrier()

    def zacc_body(j, carry):
        acc_v[pl.ds(pl.multiple_of(j * 16, 16), 16)] = zeros16
        return carry

    lax.fori_loop(0, SW // 16, zacc_body, None)

    def red_body(t, carry):
        pltpu.sync_copy(
            hist_sh.at[pl.ds(pl.multiple_of(t * HTOT + sid * SW, 128), SW)],
            tbuf_v)

        def add_body(j, c2):
            sl = pl.ds(pl.multiple_of(j * 16, 16), 16)
            acc_v[sl] = acc_v[sl] + tbuf_v[sl]
            return c2

        lax.fori_loop(0, SW // 16, add_body, None)
        return carry

    lax.fori_loop(0, NS, red_body, None)

    @pl.when(cid == 0)
    def _():
        pltpu.sync_copy(acc_v, out0_hbm.at[pl.ds(sid * SW, SW)])

    @pl.when(cid == 1)
    def _():
        pltpu.sync_copy(acc_v, out1_hbm.at[pl.ds(sid * SW, SW)])


def _matmul_body(x_ref, w_ref, d0_ref, d1_ref, o_ref):
    deg = d0_ref[...] + d1_ref[...]
    f = lax.rsqrt(jnp.maximum(deg, 1.0))
    h = jnp.dot(x_ref[...], w_ref[...], preferred_element_type=jnp.float32,
                precision=lax.Precision.HIGHEST)
    o_ref[...] = h * f


def _matmul(x_g, W, ds0, ds1):
    blk = 512
    grid = B_PAD // blk
    return pl.pallas_call(
        _matmul_body,
        grid=(grid,),
        in_specs=[
            pl.BlockSpec((blk, D), lambda i: (i, 0)),
            pl.BlockSpec((D, D), lambda i: (0, 0)),
            pl.BlockSpec((blk, 1), lambda i: (i, 0)),
            pl.BlockSpec((blk, 1), lambda i: (i, 0)),
        ],
        out_specs=pl.BlockSpec((blk, D), lambda i: (i, 0)),
        out_shape=jax.ShapeDtypeStruct((B_PAD, D), jnp.float32),
    )(x_g, W, ds0, ds1)


@functools.partial(
    pl.kernel,
    out_type=jax.ShapeDtypeStruct((NC, N_DST, D), jnp.float32),
    mesh=_mesh,
    scratch_types=[
        pltpu.VMEM((NCH, EK), jnp.int32),
        pltpu.VMEM((NCH, EK), jnp.int32),
        [pltpu.VMEM((EK, D), jnp.float32) for _ in range(6)],
        [pltpu.SemaphoreType.DMA for _ in range(6)],
        [pltpu.SemaphoreType.DMA for _ in range(6)],
        pltpu.SemaphoreType.DMA,
        pltpu.SemaphoreType.DMA,
        pltpu.VMEM_SHARED((N_DST, D), jnp.float32),
    ],
)
def _aggregate(h_hbm, esrc_hbm, edst_hbm, zero_hbm, out_hbm,
               sidx_v, didx_v, rows_bufs, gsems, ssems, isem0, isem1, agg_sh):
    cid = lax.axis_index("c")
    sid = lax.axis_index("s")
    wid = _wid()
    rpt = N_DST // NS

    id0 = pltpu.async_copy(esrc_hbm.at[wid], sidx_v, isem0)
    id1 = pltpu.async_copy(edst_hbm.at[wid], didx_v, isem1)
    pltpu.sync_copy(zero_hbm.at[pl.ds(sid * rpt, rpt), :],
                    agg_sh.at[pl.ds(sid * rpt, rpt), :])
    id0.wait()
    id1.wait()
    plsc.subcore_barrier()

    def fire_gather(j, b):
        return pltpu.async_copy(h_hbm.at[sidx_v.at[j]], rows_bufs[b],
                                gsems[b])

    def wait_gather(j, b):
        pltpu.make_async_copy(h_hbm.at[sidx_v.at[j]], rows_bufs[b],
                              gsems[b]).wait()

    def fire_scatter(j, b):
        return pltpu.async_copy(rows_bufs[b], agg_sh.at[didx_v.at[j]],
                                ssems[b], add=True)

    def wait_scatter(j, b):
        pltpu.make_async_copy(rows_bufs[b], agg_sh.at[didx_v.at[j]],
                              ssems[b]).wait()

    for b in range(3):
        fire_gather(b, b)

    NB = 20

    def body(g, carry):
        j0 = 6 * g
        for b in range(3):
            fire_gather(j0 + 3 + b, 3 + b)
        for b in range(3):
            wait_gather(j0 + b, b)
            fire_scatter(j0 + b, b)
        for b in range(3):
            wait_scatter(j0 + b, b)
            fire_gather(j0 + 6 + b, b)

        for b in range(3):
            wait_gather(j0 + 3 + b, 3 + b)
            fire_scatter(j0 + 3 + b, 3 + b)
        for b in range(3):
            wait_scatter(j0 + 3 + b, 3 + b)
        return carry

    lax.fori_loop(0, NB, body, None)
    j0 = 6 * NB
    for b in range(3):
        wait_gather(j0 + b, b)
        fire_scatter(j0 + b, b)
    fire_gather(j0 + 3, 3)
    fire_gather(j0 + 4, 4)
    for b in range(2):
        wait_gather(j0 + 3 + b, 3 + b)
        fire_scatter(j0 + 3 + b, 3 + b)
    for b in range(5):
        wait_scatter(j0 + b, b)

    plsc.subcore_barrier()
    pltpu.sync_copy(agg_sh.at[pl.ds(sid * rpt, rpt), :],
                    out_hbm.at[cid, pl.ds(sid * rpt, rpt), :])


def _final_body(p0_ref, p1_ref, d0_ref, d1_ref, b_ref, o_ref):
    g = lax.rsqrt(jnp.maximum(d0_ref[...] + d1_ref[...], 1.0))
    a = (p0_ref[...] + p1_ref[...]) * g + b_ref[...]
    o_ref[...] = jnp.where(a > 0, a, jnp.exp(jnp.minimum(a, 0.0)) - 1.0)


def _finalize(p0, p1, dd0, dd1, b2):
    return pl.pallas_call(
        _final_body,
        grid=(1,),
        in_specs=[
            pl.BlockSpec((N_DST, D), lambda i: (0, 0)),
            pl.BlockSpec((N_DST, D), lambda i: (0, 0)),
            pl.BlockSpec((N_DST, 1), lambda i: (0, 0)),
            pl.BlockSpec((N_DST, 1), lambda i: (0, 0)),
            pl.BlockSpec((1, D), lambda i: (0, 0)),
        ],
        out_specs=pl.BlockSpec((N_DST, D), lambda i: (0, 0)),
        out_shape=jax.ShapeDtypeStruct((N_DST, D), jnp.float32),
    )(p0, p1, dd0, dd1, b2)


def kernel(x, n_id, res_n_id, edge_src, edge_dst, W, b):
    del res_n_id
    nid_pad = jnp.concatenate(
        [n_id, jnp.zeros((B_PAD - N_SRC,), jnp.int32)])
    esrc_r = edge_src.reshape(NW, NCH, EK)
    edst_r = edge_dst.reshape(NW, NCH, EK)

    x_g, hist0, hist1 = _gather_hist(x, nid_pad, edge_src, edge_dst)

    ds0 = hist0[:B_PAD].reshape(B_PAD, 1)
    ds1 = hist1[:B_PAD].reshape(B_PAD, 1)
    h = _matmul(x_g, W, ds0, ds1)

    zeros2d = jnp.zeros((N_DST, D), jnp.float32)
    parts = _aggregate(h, esrc_r, edst_r, zeros2d)

    dd0 = hist0[N_SRC:N_SRC + N_DST].reshape(N_DST, 1)
    dd1 = hist1[N_SRC:N_SRC + N_DST].reshape(N_DST, 1)
    return _finalize(parts[0], parts[1], dd0, dd1, b.reshape(1, D))
